# baseline (speedup 1.0000x reference)
import functools

import jax
import jax.numpy as jnp
from jax import lax
from jax.experimental import pallas as pl
from jax.experimental.pallas import tpu as pltpu
from jax.experimental.pallas import tpu_sc as plsc

NW = 32
IDXW = 128
CH = 1024
J = CH // IDXW


def kernel(x, table):
    B, H = x.shape
    V, D = table.shape
    N = B * H
    b_per_w = N // NW
    n_ch = b_per_w // CH

    idx2d = x.reshape(N // IDXW, IDXW).astype(jnp.int32)

    mesh = plsc.VectorSubcoreMesh(core_axis_name="c", subcore_axis_name="s")

    @functools.partial(
        pl.kernel,
        mesh=mesh,
        out_type=jax.ShapeDtypeStruct((N, D), jnp.float32),
        scratch_types=[
            pltpu.VMEM((J, IDXW), jnp.int32),
            pltpu.VMEM((CH, D), jnp.float32),
            pltpu.SemaphoreType.DMA,
        ],
        compiler_params=pltpu.CompilerParams(use_tc_tiling_on_sc=False),
    )
    def gather_kernel(idx_hbm, table_hbm, out_hbm, idx_v, rows_v, sem):
        wid = lax.axis_index("s") * 2 + lax.axis_index("c")
        base = wid * b_per_w

        def body(i, carry):
            off = base + i * CH
            row_off = pl.multiple_of(off // IDXW, 8)
            pltpu.sync_copy(idx_hbm.at[pl.ds(row_off, J)], idx_v)
            copies = [
                pltpu.async_copy(
                    table_hbm.at[idx_v.at[j]],
                    rows_v.at[pl.ds(j * IDXW, IDXW)],
                    sem,
                )
                for j in range(J)
            ]
            for c in copies:
                c.wait()
            pltpu.sync_copy(rows_v, out_hbm.at[pl.ds(off, CH)])
            return carry

        lax.fori_loop(0, n_ch, body, 0)

    out = gather_kernel(idx2d, table)
    return out.reshape(B, H, D)


# speedup vs baseline: 1.0224x; 1.0224x over previous
import functools

import jax
import jax.numpy as jnp
from jax import lax
from jax.experimental import pallas as pl
from jax.experimental.pallas import tpu as pltpu
from jax.experimental.pallas import tpu_sc as plsc

NW = 32
IDXW = 128
CH = 512
J = CH // IDXW
NBUF = 3
IWIN = 4096
IROWS = IWIN // IDXW
CPW = IWIN // CH


def kernel(x, table):
    B, H = x.shape
    V, D = table.shape
    N = B * H
    b_per_w = N // NW
    n_ch = b_per_w // CH

    idx2d = x.reshape(N // IDXW, IDXW).astype(jnp.int32)

    mesh = plsc.VectorSubcoreMesh(core_axis_name="c", subcore_axis_name="s")

    @functools.partial(
        pl.kernel,
        mesh=mesh,
        out_type=jax.ShapeDtypeStruct((N, D), jnp.float32),
        scratch_types=[
            pltpu.VMEM((IROWS, IDXW), jnp.int32),
            pltpu.VMEM((NBUF, CH, D), jnp.float32),
            pltpu.SemaphoreType.DMA,
            pltpu.SemaphoreType.DMA((NBUF,)),
        ],
        compiler_params=pltpu.CompilerParams(use_tc_tiling_on_sc=False),
    )
    def gather_kernel(idx_hbm, table_hbm, out_hbm, idx_v, rows_v, gsem, ssem):
        wid = lax.axis_index("s") * 2 + lax.axis_index("c")
        base = wid * b_per_w
        base_row = wid * (b_per_w // IDXW)

        def store_desc(k, slot):
            return pltpu.make_async_copy(
                rows_v.at[slot],
                out_hbm.at[pl.ds(base + k * CH, CH)],
                ssem.at[slot],
            )

        def body(k, carry):
            slot = lax.rem(k, NBUF)

            @pl.when(lax.rem(k, CPW) == 0)
            def _():
                row_off = pl.multiple_of(base_row + (k // CPW) * IROWS, 8)
                pltpu.sync_copy(idx_hbm.at[pl.ds(row_off, IROWS)], idx_v)

            @pl.when(k >= NBUF)
            def _():
                store_desc(k - NBUF, slot).wait()

            r0 = lax.rem(k, CPW) * J
            copies = [
                pltpu.async_copy(
                    table_hbm.at[idx_v.at[r0 + j]],
                    rows_v.at[slot, pl.ds(j * IDXW, IDXW)],
                    gsem,
                )
                for j in range(J)
            ]
            for c in copies:
                c.wait()
            store_desc(k, slot).start()
            return carry

        lax.fori_loop(0, n_ch, body, 0)

        for t in range(NBUF):
            k = n_ch - NBUF + t
            store_desc(k, lax.rem(k, NBUF)).wait()

    out = gather_kernel(idx2d, table)
    return out.reshape(B, H, D)


# speedup vs baseline: 1.0316x; 1.0090x over previous
import functools

import jax
import jax.numpy as jnp
from jax import lax
from jax.experimental import pallas as pl
from jax.experimental.pallas import tpu as pltpu
from jax.experimental.pallas import tpu_sc as plsc

NW = 32
IDXW = 128
CH = 512
J = CH // IDXW
NBUF = 3
IWIN = 4096
IROWS = IWIN // IDXW
CPW = IWIN // CH


def kernel(x, table):
    B, H = x.shape
    V, D = table.shape
    N = B * H
    b_per_w = N // NW
    n_ch = b_per_w // CH

    idx2d = x.reshape(N // IDXW, IDXW).astype(jnp.int32)

    mesh = plsc.VectorSubcoreMesh(core_axis_name="c", subcore_axis_name="s")

    @functools.partial(
        pl.kernel,
        mesh=mesh,
        out_type=jax.ShapeDtypeStruct((N, D), jnp.float32),
        scratch_types=[
            pltpu.VMEM((2, IROWS, IDXW), jnp.int32),
            pltpu.VMEM((NBUF, CH, D), jnp.float32),
            pltpu.SemaphoreType.DMA,
            pltpu.SemaphoreType.DMA((NBUF,)),
        ],
        compiler_params=pltpu.CompilerParams(use_tc_tiling_on_sc=False),
    )
    def gather_kernel(idx_hbm, table_hbm, out_hbm, idx_v, rows_v, gsem, ssem):
        wid = lax.axis_index("s") * 2 + lax.axis_index("c")
        base = wid * b_per_w
        base_row = wid * (b_per_w // IDXW)

        def stage_window(w):
            row_off = pl.multiple_of(base_row + w * IROWS, 8)
            pltpu.sync_copy(idx_hbm.at[pl.ds(row_off, IROWS)], idx_v.at[lax.rem(w, 2)])

        def gather_descs(k):
            iw = lax.rem(k // CPW, 2)
            r0 = lax.rem(k, CPW) * J
            slot = lax.rem(k, NBUF)
            return [
                pltpu.make_async_copy(
                    table_hbm.at[idx_v.at[iw, r0 + j]],
                    rows_v.at[slot, pl.ds(j * IDXW, IDXW)],
                    gsem,
                )
                for j in range(J)
            ]

        def store_desc(k):
            slot = lax.rem(k, NBUF)
            return pltpu.make_async_copy(
                rows_v.at[slot],
                out_hbm.at[pl.ds(base + k * CH, CH)],
                ssem.at[slot],
            )

        stage_window(0)
        for c in gather_descs(0):
            c.start()

        def body(k, carry):
            @pl.when(k + 1 < n_ch)
            def _():
                @pl.when(lax.rem(k + 1, CPW) == 0)
                def _():
                    stage_window((k + 1) // CPW)

                @pl.when(k + 1 >= NBUF)
                def _():
                    store_desc(k + 1 - NBUF).wait()

                for c in gather_descs(k + 1):
                    c.start()

            for c in gather_descs(k):
                c.wait()
            store_desc(k).start()
            return carry

        lax.fori_loop(0, n_ch, body, 0)

        for t in range(NBUF):
            store_desc(n_ch - NBUF + t).wait()

    out = gather_kernel(idx2d, table)
    return out.reshape(B, H, D)
